# Initial kernel scaffold; baseline (speedup 1.0000x reference)
import jax
import jax.numpy as jnp
from jax.experimental import pallas as pl


def kernel(hs_pad, alloWDense, biphoneW, targets):
    raise NotImplementedError("write your pallas kernel here")


# speedup vs baseline: 3.3483x; 3.3483x over previous
import functools

import jax
import jax.numpy as jnp
from jax.experimental import pallas as pl
from jax.experimental.pallas import tpu as pltpu

NEG = -1e30

B, T, C, P, L = 32, 1024, 512, 256, 128
S = 2 * L + 1
SP = 384
TT = 256
TC = 128
NB = B // 2


def _emis_body(hs_ref, aw_ref, ext_ref, out_ref, asm_ref):
    first = (pl.program_id(1) == 0) & (pl.program_id(2) == 0)

    @pl.when(first)
    def _():
        aw = aw_ref[...]
        am = jnp.max(aw, axis=-1, keepdims=True)
        ae = jnp.exp(aw - am)
        asm_ref[...] = ae / jnp.sum(ae, axis=-1, keepdims=True)

    hs = hs_ref[0]
    m = jnp.max(hs, axis=-1, keepdims=True)
    e = jnp.exp(hs - m)
    probs = e / jnp.sum(e, axis=-1, keepdims=True)
    mixp = jax.lax.dot_general(probs, asm_ref[...],
                               (((1,), (1,)), ((), ())),
                               preferred_element_type=jnp.float32)
    ext = ext_ref[0]
    piota = jax.lax.broadcasted_iota(jnp.int32, (P, SP), 0)
    oh = jnp.where(piota == ext, 1.0, 0.0)
    mix = jax.lax.dot_general(mixp, oh,
                              (((1,), (0,)), ((), ())),
                              preferred_element_type=jnp.float32)
    em = jnp.log(jnp.maximum(mix, 1e-30))
    out_ref[...] = em.reshape(TT, 1, 1, SP)


def _scan_body(em_ref, ts_ref, tp_ref, tk_ref, out_ref, alpha_ref):
    k = pl.program_id(1)
    ts = ts_ref[...]
    tp = tp_ref[...]
    tk = tk_ref[...]

    @pl.when(k == 0)
    def _():
        em0 = em_ref[0, :, 0, :]
        lane = jax.lax.broadcasted_iota(jnp.int32, (NB, SP), 1)
        alpha_ref[...] = em0 + jnp.where(lane < 2, 0.0, NEG)

    start = jnp.where(k == 0, 1, 0)

    def step(t, alpha):
        em_t = em_ref[t, :, 0, :]
        r1 = pltpu.roll(alpha, 1, axis=1)
        r2 = pltpu.roll(alpha, 2, axis=1)
        a0 = alpha + ts
        a1 = r1 + tp
        a2 = r2 + tk
        m = jnp.maximum(jnp.maximum(a0, a1), a2)
        ssum = (jnp.exp(a0 - m) + jnp.exp(a1 - m) + jnp.exp(a2 - m))
        return em_t + m + jnp.log(ssum)

    alpha = jax.lax.fori_loop(start, TC, step, alpha_ref[...])
    alpha_ref[...] = alpha

    @pl.when(k == pl.num_programs(1) - 1)
    def _():
        fin = alpha[:, S - 2:S]
        m2 = jnp.max(fin, axis=1, keepdims=True)
        lse = m2 + jnp.log(jnp.sum(jnp.exp(fin - m2), axis=1, keepdims=True))
        out_ref[...] = jnp.broadcast_to(-lse, (NB, 128))


@jax.jit
def kernel(hs_pad, alloWDense, biphoneW, targets):
    ext = jnp.zeros((B, S), dtype=jnp.int32).at[:, 1::2].set(targets)
    ext_p1 = jnp.concatenate([ext[:, :1], ext[:, :-1]], axis=1)
    ext_p2 = jnp.concatenate([ext[:, :2], ext[:, :-2]], axis=1)
    t_stay = biphoneW[ext, ext]
    t_prev = biphoneW[ext_p1, ext].at[:, 0].set(NEG)
    s_idx = jnp.arange(S)
    skip_ok = (s_idx % 2 == 1) & (s_idx >= 2) & (ext != ext_p2)
    t_skip = jnp.where(skip_ok, biphoneW[ext_p2, ext], NEG)
    padw = ((0, 0), (0, SP - S))
    t_stay = jnp.pad(t_stay, padw, constant_values=NEG)
    t_prev = jnp.pad(t_prev, padw, constant_values=NEG)
    t_skip = jnp.pad(t_skip, padw, constant_values=NEG)
    ext3 = jnp.pad(ext, padw).reshape(B, 1, SP)

    em_s = pl.pallas_call(
        _emis_body,
        grid=(2, B // 2, T // TT),
        in_specs=[
            pl.BlockSpec((1, TT, C), lambda c, i, t: (c * (B // 2) + i, t, 0)),
            pl.BlockSpec((P, C), lambda c, i, t: (0, 0)),
            pl.BlockSpec((1, 1, SP), lambda c, i, t: (c * (B // 2) + i, 0, 0)),
        ],
        out_specs=pl.BlockSpec((TT, 1, 1, SP),
                               lambda c, i, t: (t, c * (B // 2) + i, 0, 0)),
        out_shape=jax.ShapeDtypeStruct((T, B, 1, SP), jnp.float32),
        scratch_shapes=[pltpu.VMEM((P, C), jnp.float32)],
        compiler_params=pltpu.CompilerParams(
            dimension_semantics=("parallel", "arbitrary", "arbitrary"),
        ),
    )(hs_pad, alloWDense, ext3)

    loss = pl.pallas_call(
        _scan_body,
        grid=(2, T // TC),
        in_specs=[
            pl.BlockSpec((TC, NB, 1, SP), lambda c, k: (k, c, 0, 0)),
            pl.BlockSpec((NB, SP), lambda c, k: (c, 0)),
            pl.BlockSpec((NB, SP), lambda c, k: (c, 0)),
            pl.BlockSpec((NB, SP), lambda c, k: (c, 0)),
        ],
        out_specs=pl.BlockSpec((NB, 128), lambda c, k: (c, 0)),
        out_shape=jax.ShapeDtypeStruct((B, 128), jnp.float32),
        scratch_shapes=[pltpu.VMEM((NB, SP), jnp.float32)],
        compiler_params=pltpu.CompilerParams(
            dimension_semantics=("parallel", "arbitrary"),
        ),
    )(em_s, t_stay, t_prev, t_skip)

    return jnp.mean(loss[:, 0])


# speedup vs baseline: 5.0429x; 1.5061x over previous
import functools

import jax
import jax.numpy as jnp
from jax.experimental import pallas as pl
from jax.experimental.pallas import tpu as pltpu

NEG = -1e30
LOG2E = 1.4426950408889634
LN2 = 0.6931471805599453

B, T, C, P, L = 32, 1024, 512, 256, 128
S = 2 * L + 1
SP = 384
TT = 256
TC = 128
NB = B // 2


def _emis_body(hs_ref, aw_ref, ext_ref, bw_ref, em_ref, t4_ref,
               asm_ref, oh_ref):
    t = pl.program_id(2)

    @pl.when(t == 0)
    def _():
        aw = aw_ref[...]
        am = jnp.max(aw, axis=-1, keepdims=True)
        ae = jnp.exp(aw - am)
        asm_ref[...] = (ae / jnp.sum(ae, axis=-1, keepdims=True)
                        ).astype(jnp.bfloat16)
        e = ext_ref[0]
        piota = jax.lax.broadcasted_iota(jnp.int32, (P, SP), 0)
        ohf = jnp.where(piota == e, 1.0, 0.0)
        oh_ref[...] = ohf.astype(jnp.bfloat16)
        wb = bw_ref[...].astype(jnp.bfloat16)
        g = jax.lax.dot_general(wb, oh_ref[...],
                                (((1,), (0,)), ((), ())),
                                preferred_element_type=jnp.float32)
        oh1 = pltpu.roll(ohf, 1, axis=1)
        oh2 = pltpu.roll(ohf, 2, axis=1)
        ts_v = jnp.sum(ohf * g, axis=0, keepdims=True) * LOG2E
        tp_v = jnp.sum(oh1 * g, axis=0, keepdims=True) * LOG2E
        tk_v = jnp.sum(oh2 * g, axis=0, keepdims=True) * LOG2E
        lane = jax.lax.broadcasted_iota(jnp.int32, (1, SP), 1)
        e2 = pltpu.roll(e, 2, axis=1)
        real = lane < S
        ts_row = jnp.where(real, ts_v, NEG)
        tp_row = jnp.where(real & (lane >= 1), tp_v, NEG)
        skip_ok = real & ((lane % 2) == 1) & (lane >= 2) & (e != e2)
        tk_row = jnp.where(skip_ok, tk_v, NEG)
        rows = jnp.concatenate([ts_row, tp_row, tk_row, ts_row], axis=0)
        t4_ref[...] = rows.reshape(4, 1, 1, SP)

    hs = hs_ref[0]
    m = jnp.max(hs, axis=-1, keepdims=True)
    ex = jnp.exp(hs - m)
    probs = (ex / jnp.sum(ex, axis=-1, keepdims=True)).astype(jnp.bfloat16)
    mixp = jax.lax.dot_general(probs, asm_ref[...],
                               (((1,), (1,)), ((), ())),
                               preferred_element_type=jnp.float32)
    mix = jax.lax.dot_general(mixp.astype(jnp.bfloat16), oh_ref[...],
                              (((1,), (0,)), ((), ())),
                              preferred_element_type=jnp.float32)
    em = jnp.log2(jnp.maximum(mix, 1e-30))
    em_ref[...] = em.reshape(TT, 1, 1, SP)


def _scan_body(em_ref, t4_ref, out_ref, alpha_ref, tt_ref, emc_ref):
    k = pl.program_id(0)

    emc_ref[...] = em_ref[:, :, 0, :]

    @pl.when(k == 0)
    def _():
        em0 = emc_ref[0]
        lane = jax.lax.broadcasted_iota(jnp.int32, (B, SP), 1)
        alpha_ref[...] = em0 + jnp.where(lane < 2, 0.0, NEG)
        tt_ref[0] = t4_ref[0, :, 0, :]
        tt_ref[1] = t4_ref[1, :, 0, :]
        tt_ref[2] = t4_ref[2, :, 0, :]

    start = jnp.where(k == 0, 1, 0)

    def step(t, alpha):
        em_t = emc_ref[t]
        ts = tt_ref[0]
        tp = tt_ref[1]
        tk = tt_ref[2]
        r1 = pltpu.roll(alpha, 1, axis=1)
        r2 = pltpu.roll(alpha, 2, axis=1)
        a0 = alpha + ts
        a1 = r1 + tp
        a2 = r2 + tk
        m = jnp.maximum(jnp.maximum(a0, a1), a2)
        ssum = (jnp.exp2(a0 - m) + jnp.exp2(a1 - m) + jnp.exp2(a2 - m))
        return em_t + m + jnp.log2(ssum)

    alpha = jax.lax.fori_loop(start, TC, step, alpha_ref[...])
    alpha_ref[...] = alpha

    @pl.when(k == pl.num_programs(0) - 1)
    def _():
        fin = alpha[:, S - 2:S]
        m2 = jnp.max(fin, axis=1, keepdims=True)
        lse = m2 + jnp.log2(jnp.sum(jnp.exp2(fin - m2), axis=1, keepdims=True))
        out_ref[...] = jnp.broadcast_to(-lse * LN2, (B, 128))


@jax.jit
def kernel(hs_pad, alloWDense, biphoneW, targets):
    zt = jnp.stack([jnp.zeros_like(targets), targets], axis=2).reshape(B, 2 * L)
    ext = jnp.concatenate(
        [zt, jnp.zeros((B, SP - 2 * L), dtype=targets.dtype)], axis=1)
    ext3 = ext.reshape(B, 1, SP)

    em_s, t4 = pl.pallas_call(
        _emis_body,
        grid=(2, B // 2, T // TT),
        in_specs=[
            pl.BlockSpec((1, TT, C), lambda c, i, t: (c * (B // 2) + i, t, 0)),
            pl.BlockSpec((P, C), lambda c, i, t: (0, 0)),
            pl.BlockSpec((1, 1, SP), lambda c, i, t: (c * (B // 2) + i, 0, 0)),
            pl.BlockSpec((P, P), lambda c, i, t: (0, 0)),
        ],
        out_specs=[
            pl.BlockSpec((TT, 1, 1, SP),
                         lambda c, i, t: (t, c * (B // 2) + i, 0, 0)),
            pl.BlockSpec((4, 1, 1, SP),
                         lambda c, i, t: (0, c * (B // 2) + i, 0, 0)),
        ],
        out_shape=[
            jax.ShapeDtypeStruct((T, B, 1, SP), jnp.float32),
            jax.ShapeDtypeStruct((4, B, 1, SP), jnp.float32),
        ],
        scratch_shapes=[pltpu.VMEM((P, C), jnp.bfloat16),
                        pltpu.VMEM((P, SP), jnp.bfloat16)],
        compiler_params=pltpu.CompilerParams(
            dimension_semantics=("parallel", "parallel", "arbitrary"),
        ),
    )(hs_pad, alloWDense, ext3, biphoneW)

    loss = pl.pallas_call(
        _scan_body,
        grid=(T // TC,),
        in_specs=[
            pl.BlockSpec((TC, B, 1, SP), lambda k: (k, 0, 0, 0)),
            pl.BlockSpec((4, B, 1, SP), lambda k: (0, 0, 0, 0)),
        ],
        out_specs=pl.BlockSpec((B, 128), lambda k: (0, 0)),
        out_shape=jax.ShapeDtypeStruct((B, 128), jnp.float32),
        scratch_shapes=[pltpu.VMEM((B, SP), jnp.float32),
                        pltpu.VMEM((3, B, SP), jnp.float32),
                        pltpu.VMEM((TC, B, SP), jnp.float32)],
        compiler_params=pltpu.CompilerParams(
            dimension_semantics=("arbitrary",),
        ),
    )(em_s, t4)

    return jnp.mean(loss[:, 0])
